# Initial kernel scaffold; baseline (speedup 1.0000x reference)
import jax
import jax.numpy as jnp
from jax.experimental import pallas as pl


def kernel(features, edge_index, W_self0, W_neigh0, b0, W_self1, W_neigh1, b1, W_self2, W_neigh2, b2):
    raise NotImplementedError("write your pallas kernel here")


# speedup vs baseline: 2.4624x; 2.4624x over previous
import functools

import jax
import jax.numpy as jnp
from jax import lax
from jax.experimental import pallas as pl
from jax.experimental.pallas import tpu as pltpu
from jax.experimental.pallas import tpu_sc as plsc

N = 10000
E = 320000
D = 128

NC = 2
NS = 16
NW = NC * NS
K = 128
EPW = 10240
E_PAD = NW * EPW
STEPS = EPW // K
N_PAD = 10240
ZR = N_PAD // NS

_MESH = plsc.VectorSubcoreMesh(core_axis_name="c", subcore_axis_name="s")


def _sc_agg_body(with_gather, h_hbm, src_hbm, dst_hbm, zrows_hbm, agg_out,
                 agg_sh, src_v, dst_v, rows_v, sem):
    cid = lax.axis_index("c")
    sid = lax.axis_index("s")
    wid = cid * NS + sid
    pltpu.sync_copy(zrows_hbm, agg_sh.at[pl.ds(sid * ZR, ZR)])
    if not with_gather:
        pltpu.sync_copy(h_hbm, rows_v)
    plsc.subcore_barrier()

    base = wid * EPW

    @pl.loop(0, STEPS)
    def _(i):
        off = base + i * K
        pltpu.sync_copy(dst_hbm.at[pl.ds(off, K)], dst_v)
        if with_gather:
            pltpu.sync_copy(src_hbm.at[pl.ds(off, K)], src_v)
            pltpu.async_copy(h_hbm.at[src_v], rows_v, sem).wait()
        pltpu.sync_copy(rows_v, agg_sh.at[dst_v], add=True)

    plsc.subcore_barrier()
    pltpu.sync_copy(agg_sh.at[pl.ds(sid * ZR, ZR)],
                    agg_out.at[cid, pl.ds(sid * ZR, ZR)])


def _make_sc_agg(with_gather):
    out_type = jax.ShapeDtypeStruct((NC, N_PAD, D), jnp.float32)
    scratch = [
        pltpu.VMEM_SHARED((N_PAD, D), jnp.float32),
        pltpu.VMEM((K,), jnp.int32),
        pltpu.VMEM((K,), jnp.int32),
        pltpu.VMEM((K, D), jnp.float32),
        pltpu.SemaphoreType.DMA,
    ]
    return pl.kernel(functools.partial(_sc_agg_body, with_gather),
                     out_type=out_type, mesh=_MESH, scratch_types=scratch)


_sc_agg = _make_sc_agg(True)
_sc_deg = _make_sc_agg(False)


def _rdeg_body(d_ref, o_ref):
    o_ref[...] = 1.0 / jnp.maximum(d_ref[0] + d_ref[1], 1.0)


_rdeg = pl.pallas_call(
    _rdeg_body,
    grid=(5,),
    in_specs=[pl.BlockSpec((NC, 2000, D), lambda i: (0, i, 0))],
    out_specs=pl.BlockSpec((2000, D), lambda i: (i, 0)),
    out_shape=jax.ShapeDtypeStruct((N, D), jnp.float32),
)


def _combine_body(act, h_ref, p_ref, r_ref, ws_ref, wn_ref, b_ref, o_ref):
    agg = (p_ref[0] + p_ref[1]) * r_ref[...]
    acc = jnp.dot(h_ref[...], ws_ref[...], preferred_element_type=jnp.float32)
    acc = acc + jnp.dot(agg, wn_ref[...], preferred_element_type=jnp.float32)
    acc = acc + b_ref[...]
    if act:
        acc = jnp.maximum(acc, 0.0)
    o_ref[...] = acc


def _make_combine(act, block=2000):
    return pl.pallas_call(
        functools.partial(_combine_body, act),
        grid=(N // block,),
        in_specs=[
            pl.BlockSpec((block, D), lambda i: (i, 0)),
            pl.BlockSpec((NC, block, D), lambda i: (0, i, 0)),
            pl.BlockSpec((block, D), lambda i: (i, 0)),
            pl.BlockSpec((D, D), lambda i: (0, 0)),
            pl.BlockSpec((D, D), lambda i: (0, 0)),
            pl.BlockSpec((1, D), lambda i: (0, 0)),
        ],
        out_specs=pl.BlockSpec((block, D), lambda i: (i, 0)),
        out_shape=jax.ShapeDtypeStruct((N, D), jnp.float32),
    )


_combine_relu = _make_combine(True)
_combine_lin = _make_combine(False)


def kernel(features, edge_index, W_self0, W_neigh0, b0, W_self1, W_neigh1,
           b1, W_self2, W_neigh2, b2):
    src = edge_index[0].astype(jnp.int32)
    dst = edge_index[1].astype(jnp.int32)
    pad = E_PAD - E
    srcp = jnp.concatenate([src, jnp.zeros((pad,), jnp.int32)])
    dstp = jnp.concatenate([dst, jnp.full((pad,), N, jnp.int32)])
    zrows = jnp.zeros((ZR, D), jnp.float32)
    ones_rows = jnp.ones((K, D), jnp.float32)

    degp = _sc_deg(ones_rows, srcp, dstp, zrows)
    rdeg = _rdeg(degp)
    agg0 = _sc_agg(features, srcp, dstp, zrows)
    h1 = _combine_relu(features, agg0, rdeg, W_self0, W_neigh0,
                       b0.reshape(1, D))
    agg1 = _sc_agg(h1, srcp, dstp, zrows)
    h2 = _combine_relu(h1, agg1, rdeg, W_self1, W_neigh1, b1.reshape(1, D))
    agg2 = _sc_agg(h2, srcp, dstp, zrows)
    h3 = _combine_lin(h2, agg2, rdeg, W_self2, W_neigh2, b2.reshape(1, D))
    return h3


# speedup vs baseline: 3.1173x; 1.2659x over previous
import functools

import jax
import jax.numpy as jnp
from jax import lax
from jax.experimental import pallas as pl
from jax.experimental.pallas import tpu as pltpu
from jax.experimental.pallas import tpu_sc as plsc

N = 10000
E = 320000
D = 128

NC = 2
NS = 16
NW = NC * NS
K = 80
EPW = 10240
E_PAD = NW * EPW
STEPS = EPW // K
N_PAD = 10112
ZR = N_PAD // NS

_MESH = plsc.VectorSubcoreMesh(core_axis_name="c", subcore_axis_name="s")


def _sc_agg_body(with_gather, h_hbm, src_hbm, dst_hbm, zrows_hbm, agg_out,
                 agg_sh, src_sl, dst_a, dst_b, buf_a, buf_b, semg, semd):
    cid = lax.axis_index("c")
    sid = lax.axis_index("s")
    wid = cid * NS + sid
    pltpu.sync_copy(zrows_hbm, agg_sh.at[pl.ds(sid * ZR, ZR)])
    if with_gather:
        pltpu.sync_copy(src_hbm.at[wid], src_sl)
    else:
        pltpu.sync_copy(h_hbm, buf_a)
    plsc.subcore_barrier()

    def fire_dst(i, dbuf):
        pltpu.async_copy(dst_hbm.at[wid, i], dbuf, semd)

    def wait_dst(i, dbuf):
        pltpu.make_async_copy(dst_hbm.at[wid, i], dbuf, semd).wait()

    def fire_g(i, buf):
        pltpu.async_copy(h_hbm.at[src_sl.at[i]], buf, semg)

    def wait_g(i, buf):
        pltpu.make_async_copy(h_hbm.at[src_sl.at[i]], buf, semg).wait()

    if with_gather:
        fire_dst(0, dst_a)
        fire_g(0, buf_a)

        @pl.loop(0, STEPS // 2)
        def _(t):
            i0 = 2 * t
            i1 = i0 + 1
            i2 = i0 + 2
            fire_dst(i1, dst_b)
            fire_g(i1, buf_b)
            wait_g(i0, buf_a)
            wait_dst(i0, dst_a)
            pltpu.sync_copy(buf_a, agg_sh.at[dst_a], add=True)

            @pl.when(i2 < STEPS)
            def _():
                fire_dst(i2, dst_a)
                fire_g(i2, buf_a)

            wait_g(i1, buf_b)
            wait_dst(i1, dst_b)
            pltpu.sync_copy(buf_b, agg_sh.at[dst_b], add=True)
    else:
        fire_dst(0, dst_a)

        @pl.loop(0, STEPS // 2)
        def _(t):
            i0 = 2 * t
            i1 = i0 + 1
            i2 = i0 + 2
            fire_dst(i1, dst_b)
            wait_dst(i0, dst_a)
            pltpu.sync_copy(buf_a, agg_sh.at[dst_a], add=True)

            @pl.when(i2 < STEPS)
            def _():
                fire_dst(i2, dst_a)

            wait_dst(i1, dst_b)
            pltpu.sync_copy(buf_a, agg_sh.at[dst_b], add=True)

    plsc.subcore_barrier()
    pltpu.sync_copy(agg_sh.at[pl.ds(sid * ZR, ZR)],
                    agg_out.at[cid, pl.ds(sid * ZR, ZR)])


def _make_sc_agg(with_gather):
    out_type = jax.ShapeDtypeStruct((NC, N_PAD, D), jnp.float32)
    scratch = [
        pltpu.VMEM_SHARED((N_PAD, D), jnp.float32),
        pltpu.VMEM((STEPS, K), jnp.int32),
        pltpu.VMEM((K,), jnp.int32),
        pltpu.VMEM((K,), jnp.int32),
        pltpu.VMEM((K, D), jnp.float32),
        pltpu.VMEM((K, D), jnp.float32),
        pltpu.SemaphoreType.DMA,
        pltpu.SemaphoreType.DMA,
    ]
    return pl.kernel(functools.partial(_sc_agg_body, with_gather),
                     out_type=out_type, mesh=_MESH, scratch_types=scratch)


_sc_agg = _make_sc_agg(True)
_sc_deg = _make_sc_agg(False)


def _rdeg_body(d_ref, o_ref):
    o_ref[...] = 1.0 / jnp.maximum(d_ref[0] + d_ref[1], 1.0)


_rdeg = pl.pallas_call(
    _rdeg_body,
    grid=(5,),
    in_specs=[pl.BlockSpec((NC, 2000, D), lambda i: (0, i, 0))],
    out_specs=pl.BlockSpec((2000, D), lambda i: (i, 0)),
    out_shape=jax.ShapeDtypeStruct((N, D), jnp.float32),
)


def _combine_body(act, h_ref, p_ref, r_ref, ws_ref, wn_ref, b_ref, o_ref):
    agg = (p_ref[0] + p_ref[1]) * r_ref[...]
    acc = jnp.dot(h_ref[...], ws_ref[...], preferred_element_type=jnp.float32)
    acc = acc + jnp.dot(agg, wn_ref[...], preferred_element_type=jnp.float32)
    acc = acc + b_ref[...]
    if act:
        acc = jnp.maximum(acc, 0.0)
    o_ref[...] = acc


def _make_combine(act, block=2000):
    return pl.pallas_call(
        functools.partial(_combine_body, act),
        grid=(N // block,),
        in_specs=[
            pl.BlockSpec((block, D), lambda i: (i, 0)),
            pl.BlockSpec((NC, block, D), lambda i: (0, i, 0)),
            pl.BlockSpec((block, D), lambda i: (i, 0)),
            pl.BlockSpec((D, D), lambda i: (0, 0)),
            pl.BlockSpec((D, D), lambda i: (0, 0)),
            pl.BlockSpec((1, D), lambda i: (0, 0)),
        ],
        out_specs=pl.BlockSpec((block, D), lambda i: (i, 0)),
        out_shape=jax.ShapeDtypeStruct((N, D), jnp.float32),
    )


_combine_relu = _make_combine(True)
_combine_lin = _make_combine(False)


def kernel(features, edge_index, W_self0, W_neigh0, b0, W_self1, W_neigh1,
           b1, W_self2, W_neigh2, b2):
    src = edge_index[0].astype(jnp.int32)
    dst = edge_index[1].astype(jnp.int32)
    pad = E_PAD - E
    srcp = jnp.concatenate([src, jnp.zeros((pad,), jnp.int32)])
    dstp = jnp.concatenate([dst, jnp.full((pad,), N, jnp.int32)])
    srcp = srcp.reshape(NW, STEPS, K)
    dstp = dstp.reshape(NW, STEPS, K)
    zrows = jnp.zeros((ZR, D), jnp.float32)
    ones_rows = jnp.ones((K, D), jnp.float32)

    degp = _sc_deg(ones_rows, srcp, dstp, zrows)
    rdeg = _rdeg(degp)
    agg0 = _sc_agg(features, srcp, dstp, zrows)
    h1 = _combine_relu(features, agg0, rdeg, W_self0, W_neigh0,
                       b0.reshape(1, D))
    agg1 = _sc_agg(h1, srcp, dstp, zrows)
    h2 = _combine_relu(h1, agg1, rdeg, W_self1, W_neigh1, b1.reshape(1, D))
    agg2 = _sc_agg(h2, srcp, dstp, zrows)
    h3 = _combine_lin(h2, agg2, rdeg, W_self2, W_neigh2, b2.reshape(1, D))
    return h3


# speedup vs baseline: 3.1419x; 1.0079x over previous
import functools

import jax
import jax.numpy as jnp
from jax import lax
from jax.experimental import pallas as pl
from jax.experimental.pallas import tpu as pltpu
from jax.experimental.pallas import tpu_sc as plsc

N = 10000
E = 320000
D = 128

NC = 2
NS = 16
NW = NC * NS
K = 64
NBUF = 3
EPW = 10240
E_PAD = NW * EPW
STEPS = EPW // K
N_PAD = 10112
ZR = N_PAD // NS

_MESH = plsc.VectorSubcoreMesh(core_axis_name="c", subcore_axis_name="s")


def _sc_agg_body(with_gather, h_hbm, src_hbm, dst_hbm, zrows_hbm, agg_out,
                 agg_sh, src_sl, dst_0, dst_1, dst_2, buf_0, buf_1, buf_2,
                 semg, semd, sems):
    cid = lax.axis_index("c")
    sid = lax.axis_index("s")
    wid = cid * NS + sid
    bufs = [buf_0, buf_1, buf_2]
    dsts = [dst_0, dst_1, dst_2]
    pltpu.sync_copy(zrows_hbm, agg_sh.at[pl.ds(sid * ZR, ZR)])
    if with_gather:
        pltpu.sync_copy(src_hbm.at[wid], src_sl)
    else:
        pltpu.sync_copy(h_hbm, buf_0)
    plsc.subcore_barrier()

    def fire_dst(i, j):
        pltpu.async_copy(dst_hbm.at[wid, i], dsts[j], semd)

    def wait_dst(i, j):
        pltpu.make_async_copy(dst_hbm.at[wid, i], dsts[j], semd).wait()

    def fire_g(i, j):
        pltpu.async_copy(h_hbm.at[src_sl.at[i]], bufs[j], semg)

    def wait_g(i, j):
        pltpu.make_async_copy(h_hbm.at[src_sl.at[i]], bufs[j], semg).wait()

    def fire_s(j):
        pltpu.async_copy(bufs[j], agg_sh.at[dsts[j]], sems, add=True)

    def wait_s_one():
        pltpu.make_async_copy(bufs[0], agg_sh.at[dsts[0]], sems).wait()

    if with_gather:
        fire_dst(0, 0)
        fire_g(0, 0)
        fire_dst(1, 1)
        fire_g(1, 1)

        def step(i, j):
            @pl.when(i + 2 < STEPS)
            def _():
                @pl.when(i >= 1)
                def _():
                    wait_s_one()
                fire_dst(i + 2, (j + 2) % NBUF)
                fire_g(i + 2, (j + 2) % NBUF)

            wait_g(i, j)
            wait_dst(i, j)
            fire_s(j)

        @pl.loop(0, (STEPS - 1) // NBUF)
        def _(t):
            i0 = NBUF * t
            for j in range(NBUF):
                step(i0 + j, j)

        step(STEPS - 1, (STEPS - 1) % NBUF)
        wait_s_one()
        wait_s_one()
        wait_s_one()
    else:
        fire_dst(0, 0)

        @pl.loop(0, STEPS // 2)
        def _(t):
            i0 = 2 * t
            i1 = i0 + 1
            i2 = i0 + 2
            fire_dst(i1, 1)
            wait_dst(i0, 0)
            pltpu.sync_copy(buf_0, agg_sh.at[dst_0], add=True)

            @pl.when(i2 < STEPS)
            def _():
                fire_dst(i2, 0)

            wait_dst(i1, 1)
            pltpu.sync_copy(buf_0, agg_sh.at[dst_1], add=True)

    plsc.subcore_barrier()
    pltpu.sync_copy(agg_sh.at[pl.ds(sid * ZR, ZR)],
                    agg_out.at[cid, pl.ds(sid * ZR, ZR)])


def _make_sc_agg(with_gather):
    out_type = jax.ShapeDtypeStruct((NC, N_PAD, D), jnp.float32)
    scratch = [
        pltpu.VMEM_SHARED((N_PAD, D), jnp.float32),
        pltpu.VMEM((STEPS, K), jnp.int32),
        pltpu.VMEM((K,), jnp.int32),
        pltpu.VMEM((K,), jnp.int32),
        pltpu.VMEM((K,), jnp.int32),
        pltpu.VMEM((K, D), jnp.float32),
        pltpu.VMEM((K, D), jnp.float32),
        pltpu.VMEM((K, D), jnp.float32),
        pltpu.SemaphoreType.DMA,
        pltpu.SemaphoreType.DMA,
        pltpu.SemaphoreType.DMA,
    ]
    return pl.kernel(functools.partial(_sc_agg_body, with_gather),
                     out_type=out_type, mesh=_MESH, scratch_types=scratch)


_sc_agg = _make_sc_agg(True)
_sc_deg = _make_sc_agg(False)


def _rdeg_body(d_ref, o_ref):
    o_ref[...] = 1.0 / jnp.maximum(d_ref[0] + d_ref[1], 1.0)


_rdeg = pl.pallas_call(
    _rdeg_body,
    grid=(5,),
    in_specs=[pl.BlockSpec((NC, 2000, D), lambda i: (0, i, 0))],
    out_specs=pl.BlockSpec((2000, D), lambda i: (i, 0)),
    out_shape=jax.ShapeDtypeStruct((N, D), jnp.float32),
)


def _combine_body(act, h_ref, p_ref, r_ref, ws_ref, wn_ref, b_ref, o_ref):
    agg = (p_ref[0] + p_ref[1]) * r_ref[...]
    acc = jnp.dot(h_ref[...], ws_ref[...], preferred_element_type=jnp.float32)
    acc = acc + jnp.dot(agg, wn_ref[...], preferred_element_type=jnp.float32)
    acc = acc + b_ref[...]
    if act:
        acc = jnp.maximum(acc, 0.0)
    o_ref[...] = acc


def _make_combine(act, block=2000):
    return pl.pallas_call(
        functools.partial(_combine_body, act),
        grid=(N // block,),
        in_specs=[
            pl.BlockSpec((block, D), lambda i: (i, 0)),
            pl.BlockSpec((NC, block, D), lambda i: (0, i, 0)),
            pl.BlockSpec((block, D), lambda i: (i, 0)),
            pl.BlockSpec((D, D), lambda i: (0, 0)),
            pl.BlockSpec((D, D), lambda i: (0, 0)),
            pl.BlockSpec((1, D), lambda i: (0, 0)),
        ],
        out_specs=pl.BlockSpec((block, D), lambda i: (i, 0)),
        out_shape=jax.ShapeDtypeStruct((N, D), jnp.float32),
    )


_combine_relu = _make_combine(True)
_combine_lin = _make_combine(False)


def kernel(features, edge_index, W_self0, W_neigh0, b0, W_self1, W_neigh1,
           b1, W_self2, W_neigh2, b2):
    src = edge_index[0].astype(jnp.int32)
    dst = edge_index[1].astype(jnp.int32)
    pad = E_PAD - E
    srcp = jnp.concatenate([src, jnp.zeros((pad,), jnp.int32)])
    dstp = jnp.concatenate([dst, jnp.full((pad,), N, jnp.int32)])
    srcp = srcp.reshape(NW, STEPS, K)
    dstp = dstp.reshape(NW, STEPS, K)
    zrows = jnp.zeros((ZR, D), jnp.float32)
    ones_rows = jnp.ones((K, D), jnp.float32)

    degp = _sc_deg(ones_rows, srcp, dstp, zrows)
    rdeg = _rdeg(degp)
    agg0 = _sc_agg(features, srcp, dstp, zrows)
    h1 = _combine_relu(features, agg0, rdeg, W_self0, W_neigh0,
                       b0.reshape(1, D))
    agg1 = _sc_agg(h1, srcp, dstp, zrows)
    h2 = _combine_relu(h1, agg1, rdeg, W_self1, W_neigh1, b1.reshape(1, D))
    agg2 = _sc_agg(h2, srcp, dstp, zrows)
    h3 = _combine_lin(h2, agg2, rdeg, W_self2, W_neigh2, b2.reshape(1, D))
    return h3


# speedup vs baseline: 3.4129x; 1.0863x over previous
import functools

import jax
import jax.numpy as jnp
from jax import lax
from jax.experimental import pallas as pl
from jax.experimental.pallas import tpu as pltpu
from jax.experimental.pallas import tpu_sc as plsc

N = 10000
E = 320000
D = 128

NC = 2
NS = 16
NW = NC * NS
K = 64
EPW = 10240
E_PAD = NW * EPW
STEPS = EPW // K
EPW0 = 15360
EPW1 = EPW * 2 - EPW0
STEPS0 = EPW0 // K
STEPS1 = EPW1 // K
N_PAD = 10112
ZR = N_PAD // NS

_MESH = plsc.VectorSubcoreMesh(core_axis_name="c", subcore_axis_name="s")


def _sc_agg_body(with_gather, h_hbm, src_hbm, dst_hbm, zrows_hbm, agg_out,
                 agg_sh, src_sl, dst_0, dst_1, buf_0, buf_1, semg, semd):
    cid = lax.axis_index("c")
    sid = lax.axis_index("s")
    wid = cid * NS + sid
    bufs = [buf_0, buf_1]
    dsts = [dst_0, dst_1]
    pltpu.sync_copy(zrows_hbm, agg_sh.at[pl.ds(sid * ZR, ZR)])
    if not with_gather:
        pltpu.sync_copy(h_hbm, buf_0)
    plsc.subcore_barrier()

    def fire_dst(ebase, i, j):
        pltpu.async_copy(dst_hbm.at[pl.ds(ebase + i * K, K)], dsts[j], semd)

    def wait_dst(ebase, i, j):
        pltpu.make_async_copy(dst_hbm.at[pl.ds(ebase + i * K, K)],
                              dsts[j], semd).wait()

    def fire_g(i, j):
        pltpu.async_copy(h_hbm.at[src_sl.at[pl.ds(i * K, K)]], bufs[j], semg)

    def wait_g(i, j):
        pltpu.make_async_copy(h_hbm.at[src_sl.at[pl.ds(i * K, K)]],
                              bufs[j], semg).wait()

    def gather_pipe(nsteps, ebase):
        pltpu.sync_copy(src_hbm.at[pl.ds(ebase, nsteps * K)],
                        src_sl.at[pl.ds(0, nsteps * K)])
        fire_dst(ebase, 0, 0)
        fire_g(0, 0)

        @pl.loop(0, nsteps // 2)
        def _(t):
            i0 = 2 * t
            i1 = i0 + 1
            i2 = i0 + 2
            fire_dst(ebase, i1, 1)
            fire_g(i1, 1)
            wait_g(i0, 0)
            wait_dst(ebase, i0, 0)
            pltpu.sync_copy(buf_0, agg_sh.at[dst_0], add=True)

            @pl.when(i2 < nsteps)
            def _():
                fire_dst(ebase, i2, 0)
                fire_g(i2, 0)

            wait_g(i1, 1)
            wait_dst(ebase, i1, 1)
            pltpu.sync_copy(buf_1, agg_sh.at[dst_1], add=True)

    if with_gather:
        @pl.when(cid == 0)
        def _():
            gather_pipe(STEPS0, sid * EPW0)

        @pl.when(cid == 1)
        def _():
            gather_pipe(STEPS1, NS * EPW0 + sid * EPW1)
    else:
        ebase = wid * EPW
        fire_dst(ebase, 0, 0)

        @pl.loop(0, STEPS // 2)
        def _(t):
            i0 = 2 * t
            i1 = i0 + 1
            i2 = i0 + 2
            fire_dst(ebase, i1, 1)
            wait_dst(ebase, i0, 0)
            pltpu.sync_copy(buf_0, agg_sh.at[dst_0], add=True)

            @pl.when(i2 < STEPS)
            def _():
                fire_dst(ebase, i2, 0)

            wait_dst(ebase, i1, 1)
            pltpu.sync_copy(buf_0, agg_sh.at[dst_1], add=True)

    plsc.subcore_barrier()
    pltpu.sync_copy(agg_sh.at[pl.ds(sid * ZR, ZR)],
                    agg_out.at[cid, pl.ds(sid * ZR, ZR)])


def _make_sc_agg(with_gather):
    out_type = jax.ShapeDtypeStruct((NC, N_PAD, D), jnp.float32)
    scratch = [
        pltpu.VMEM_SHARED((N_PAD, D), jnp.float32),
        pltpu.VMEM((EPW0,), jnp.int32),
        pltpu.VMEM((K,), jnp.int32),
        pltpu.VMEM((K,), jnp.int32),
        pltpu.VMEM((K, D), jnp.float32),
        pltpu.VMEM((K, D), jnp.float32),
        pltpu.SemaphoreType.DMA,
        pltpu.SemaphoreType.DMA,
    ]
    return pl.kernel(functools.partial(_sc_agg_body, with_gather),
                     out_type=out_type, mesh=_MESH, scratch_types=scratch)


_sc_agg = _make_sc_agg(True)
_sc_deg = _make_sc_agg(False)


def _rdeg_body(d_ref, o_ref):
    o_ref[...] = 1.0 / jnp.maximum(d_ref[0] + d_ref[1], 1.0)


_rdeg = pl.pallas_call(
    _rdeg_body,
    grid=(5,),
    in_specs=[pl.BlockSpec((NC, 2000, D), lambda i: (0, i, 0))],
    out_specs=pl.BlockSpec((2000, D), lambda i: (i, 0)),
    out_shape=jax.ShapeDtypeStruct((N, D), jnp.float32),
)


def _combine_body(act, h_ref, p_ref, r_ref, ws_ref, wn_ref, b_ref, o_ref):
    agg = (p_ref[0] + p_ref[1]) * r_ref[...]
    acc = jnp.dot(h_ref[...], ws_ref[...], preferred_element_type=jnp.float32)
    acc = acc + jnp.dot(agg, wn_ref[...], preferred_element_type=jnp.float32)
    acc = acc + b_ref[...]
    if act:
        acc = jnp.maximum(acc, 0.0)
    o_ref[...] = acc


def _make_combine(act, block=2000):
    return pl.pallas_call(
        functools.partial(_combine_body, act),
        grid=(N // block,),
        in_specs=[
            pl.BlockSpec((block, D), lambda i: (i, 0)),
            pl.BlockSpec((NC, block, D), lambda i: (0, i, 0)),
            pl.BlockSpec((block, D), lambda i: (i, 0)),
            pl.BlockSpec((D, D), lambda i: (0, 0)),
            pl.BlockSpec((D, D), lambda i: (0, 0)),
            pl.BlockSpec((1, D), lambda i: (0, 0)),
        ],
        out_specs=pl.BlockSpec((block, D), lambda i: (i, 0)),
        out_shape=jax.ShapeDtypeStruct((N, D), jnp.float32),
    )


_combine_relu = _make_combine(True)
_combine_lin = _make_combine(False)


def kernel(features, edge_index, W_self0, W_neigh0, b0, W_self1, W_neigh1,
           b1, W_self2, W_neigh2, b2):
    src = edge_index[0].astype(jnp.int32)
    dst = edge_index[1].astype(jnp.int32)
    pad = E_PAD - E
    srcp = jnp.concatenate([src, jnp.zeros((pad,), jnp.int32)])
    dstp = jnp.concatenate([dst, jnp.full((pad,), N, jnp.int32)])
    zrows = jnp.zeros((ZR, D), jnp.float32)
    ones_rows = jnp.ones((K, D), jnp.float32)

    degp = _sc_deg(ones_rows, srcp, dstp, zrows)
    rdeg = _rdeg(degp)
    agg0 = _sc_agg(features, srcp, dstp, zrows)
    h1 = _combine_relu(features, agg0, rdeg, W_self0, W_neigh0,
                       b0.reshape(1, D))
    agg1 = _sc_agg(h1, srcp, dstp, zrows)
    h2 = _combine_relu(h1, agg1, rdeg, W_self1, W_neigh1, b1.reshape(1, D))
    agg2 = _sc_agg(h2, srcp, dstp, zrows)
    h3 = _combine_lin(h2, agg2, rdeg, W_self2, W_neigh2, b2.reshape(1, D))
    return h3
